# baseline (speedup 1.0000x reference)
import functools
import math

import jax
import jax.numpy as jnp
from jax import lax
from jax.experimental import pallas as pl
from jax.experimental.pallas import tpu as pltpu
from jax.experimental.pallas import tpu_sc as plsc

_TOP_K = 2
_CAP_FACTOR = 1.25


def _pack_body(cap, E, TB, lg_ref, sc0_ref, sc1_ref, sg0_ref, sg1_ref,
               w0_ref, w1_ref, cnt_ref, carry_ref):
    i = pl.program_id(0)

    @pl.when(i == 0)
    def _init():
        carry_ref[...] = jnp.zeros_like(carry_ref)

    lg = lg_ref[0]
    sub = jax.lax.broadcasted_iota(jnp.int32, (E, TB), 0)

    v0 = jnp.max(lg, axis=0, keepdims=True)
    i0 = jnp.min(jnp.where(lg == v0, sub, E), axis=0, keepdims=True)
    ex0 = (sub == i0)
    lg1 = jnp.where(ex0, -jnp.inf, lg)
    v1 = jnp.max(lg1, axis=0, keepdims=True)
    i1 = jnp.min(jnp.where(lg1 == v1, sub, E), axis=0, keepdims=True)
    ex1 = (sub == i1)

    m = jnp.maximum(v0, v1)
    e0 = jnp.exp(v0 - m)
    e1 = jnp.exp(v1 - m)
    den = e0 + e1 + 1e-12
    p0 = e0 / den
    p1 = e1 / den

    ex0f = ex0.astype(jnp.float32)
    ex1f = ex1.astype(jnp.float32)
    comb = ex0f + ex1f
    r = jax.lax.broadcasted_iota(jnp.int32, (TB, TB), 0)
    c = jax.lax.broadcasted_iota(jnp.int32, (TB, TB), 1)
    U = (r < c).astype(jnp.float32)
    excl = jax.lax.dot_general(comb, U, (((1,), (0,)), ((), ())),
                               preferred_element_type=jnp.float32)
    tot = excl + carry_ref[:, :1]
    rank0 = jnp.sum(ex0f * tot, axis=0, keepdims=True).astype(jnp.int32)
    rank1 = jnp.sum(ex1f * tot, axis=0, keepdims=True).astype(jnp.int32)
    carry_ref[:, :1] += jnp.sum(comb, axis=1, keepdims=True)

    trash = E * cap
    slot0 = i0 * cap + rank0
    slot1 = i1 * cap + rank1
    k0 = rank0 < cap
    k1 = rank1 < cap
    sc0_ref[...] = jnp.where(k0, slot0, trash).reshape(1, 1, TB)
    sc1_ref[...] = jnp.where(k1, slot1, trash).reshape(1, 1, TB)
    sg0_ref[...] = jnp.where(k0, slot0, 0).reshape(1, 1, TB)
    sg1_ref[...] = jnp.where(k1, slot1, 0).reshape(1, 1, TB)
    w0_ref[...] = jnp.where(k0, p0, 0.0).reshape(1, 1, TB)
    w1_ref[...] = jnp.where(k1, p1, 0.0).reshape(1, 1, TB)
    cnt_ref[...] = carry_ref[...]


def _pack(logits_t, cap, token_block=256):
    E, T = logits_t.shape
    TB = min(token_block, T)
    nb = T // TB
    i32 = jax.ShapeDtypeStruct((nb, 1, TB), jnp.int32)
    f32 = jax.ShapeDtypeStruct((nb, 1, TB), jnp.float32)
    ospec = pl.BlockSpec((1, 1, TB), lambda i: (i, 0, 0))
    outs = pl.pallas_call(
        functools.partial(_pack_body, cap, E, TB),
        grid=(nb,),
        in_specs=[pl.BlockSpec((1, E, TB), lambda i: (0, 0, i))],
        out_specs=[ospec] * 6 + [pl.BlockSpec((E, 128), lambda i: (0, 0))],
        out_shape=[i32, i32, i32, i32, f32, f32,
                   jax.ShapeDtypeStruct((E, 128), jnp.float32)],
        scratch_shapes=[pltpu.VMEM((E, 128), jnp.float32)],
        compiler_params=pltpu.CompilerParams(
            dimension_semantics=("arbitrary",)),
    )(logits_t.reshape(1, E, T))
    return [o.reshape(T) for o in outs[:6]] + [outs[6][:, 0].astype(jnp.int32)]


def _invert_slots(sc0, sc1, n_slots):
    T = sc0.shape[0]
    n_pad = n_slots + 8
    mesh = plsc.VectorSubcoreMesh(core_axis_name="c", subcore_axis_name="s")
    nw = 32
    per = T // nw

    @functools.partial(
        pl.kernel, mesh=mesh,
        out_type=jax.ShapeDtypeStruct((n_pad,), jnp.int32),
        scratch_types=[
            pltpu.VMEM((per,), jnp.int32),
            pltpu.VMEM((per,), jnp.int32),
            pltpu.SemaphoreType.DMA,
        ],
    )
    def k(sc0_hbm, sc1_hbm, out_hbm, idx_v, val_v, sem):
        wid = lax.axis_index("s") * 2 + lax.axis_index("c")
        base = wid * per
        for i in range(per // 16):
            val_v[pl.ds(i * 16, 16)] = lax.iota(jnp.int32, 16) + (base + i * 16)
        pltpu.sync_copy(sc0_hbm.at[pl.ds(base, per)], idx_v)
        pltpu.async_copy(val_v, out_hbm.at[idx_v], sem).wait()
        pltpu.sync_copy(sc1_hbm.at[pl.ds(base, per)], idx_v)
        pltpu.async_copy(val_v, out_hbm.at[idx_v], sem).wait()

    return k(sc0, sc1)[:n_slots]


def _ffn_body(RB, D, cnt_ref, x_ref, w1_ref, w2_ref, o_ref):
    e = pl.program_id(0)
    j = pl.program_id(1)
    rb = pl.program_id(2)
    rows = pl.ds(rb * RB, RB)
    active = cnt_ref[e] > rb * RB

    @pl.when(active)
    def _compute():
        xb = x_ref[0].astype(jnp.bfloat16)
        w1 = w1_ref[0].astype(jnp.bfloat16)
        w2 = w2_ref[0].astype(jnp.bfloat16)
        h = jax.lax.dot_general(xb, w1, (((1,), (1,)), ((), ())),
                                preferred_element_type=jnp.float32)
        h = 0.5 * h * (1.0 + jax.lax.erf(h * 0.7071067811865476))
        y = jax.lax.dot_general(h.astype(jnp.bfloat16), w2,
                                (((1,), (1,)), ((), ())),
                                preferred_element_type=jnp.float32)

        @pl.when(j == 0)
        def _init():
            o_ref[0, rows, :] = y

        @pl.when(j != 0)
        def _acc():
            o_ref[0, rows, :] += y

    @pl.when(jnp.logical_and(jnp.logical_not(active), j == 0))
    def _zero():
        o_ref[0, rows, :] = jnp.zeros((RB, D), jnp.float32)


def _expert_ffn(counts, xbuf, fc1, fc2, block_ff=1024, block_rows=256):
    E, cap, D = xbuf.shape
    DFF = fc1.shape[1]
    block_ff = min(block_ff, DFF)
    nj = DFF // block_ff
    RB = min(block_rows, cap)
    nrb = pl.cdiv(cap, RB)
    grid_spec = pltpu.PrefetchScalarGridSpec(
        num_scalar_prefetch=1,
        grid=(E, nj, nrb),
        in_specs=[
            pl.BlockSpec((1, RB, D), lambda e, j, rb, *_: (e, rb, 0)),
            pl.BlockSpec((1, block_ff, D), lambda e, j, rb, *_: (e, j, 0)),
            pl.BlockSpec((1, D, block_ff), lambda e, j, rb, *_: (e, 0, j)),
        ],
        out_specs=pl.BlockSpec((1, cap, D), lambda e, j, rb, *_: (e, 0, 0)),
    )
    return pl.pallas_call(
        functools.partial(_ffn_body, RB, D),
        grid_spec=grid_spec,
        out_shape=jax.ShapeDtypeStruct((E, cap, D), jnp.float32),
        compiler_params=pltpu.CompilerParams(
            dimension_semantics=("arbitrary", "arbitrary", "arbitrary")),
    )(counts, xbuf, fc1, fc2)


def kernel(x, W_router, fc1, fc2):
    T, D = x.shape
    E = W_router.shape[0]
    S = T * _TOP_K
    cap = max(1, math.ceil(S * _CAP_FACTOR / E))

    logits = x @ W_router.T
    sc0, sc1, sg0, sg1, w0, w1, counts = _pack(logits.T, cap)

    src = _invert_slots(sc0, sc1, E * cap)
    xbuf = x[src].reshape(E, cap, D)

    y = _expert_ffn(counts, xbuf, fc1, fc2).reshape(E * cap, D)

    return y[sg0] * w0[:, None] + y[sg1] * w1[:, None]


# speedup vs baseline: 1.2502x; 1.2502x over previous
import functools
import math

import jax
import jax.numpy as jnp
from jax import lax
from jax.experimental import pallas as pl
from jax.experimental.pallas import tpu as pltpu
from jax.experimental.pallas import tpu_sc as plsc

_TOP_K = 2
_CAP_FACTOR = 1.25


def _pack_body(cap, E, TB, lg_ref, sc0_ref, sc1_ref, sg0_ref, sg1_ref,
               w0_ref, w1_ref, cnt_ref, carry_ref):
    i = pl.program_id(0)

    @pl.when(i == 0)
    def _init():
        carry_ref[...] = jnp.zeros_like(carry_ref)

    lg = lg_ref[0]
    sub = jax.lax.broadcasted_iota(jnp.int32, (E, TB), 0)

    v0 = jnp.max(lg, axis=0, keepdims=True)
    i0 = jnp.min(jnp.where(lg == v0, sub, E), axis=0, keepdims=True)
    ex0 = (sub == i0)
    lg1 = jnp.where(ex0, -jnp.inf, lg)
    v1 = jnp.max(lg1, axis=0, keepdims=True)
    i1 = jnp.min(jnp.where(lg1 == v1, sub, E), axis=0, keepdims=True)
    ex1 = (sub == i1)

    m = jnp.maximum(v0, v1)
    e0 = jnp.exp(v0 - m)
    e1 = jnp.exp(v1 - m)
    den = e0 + e1 + 1e-12
    p0 = e0 / den
    p1 = e1 / den

    ex0f = ex0.astype(jnp.float32)
    ex1f = ex1.astype(jnp.float32)
    comb = ex0f + ex1f
    r = jax.lax.broadcasted_iota(jnp.int32, (TB, TB), 0)
    c = jax.lax.broadcasted_iota(jnp.int32, (TB, TB), 1)
    U = (r < c).astype(jnp.float32)
    excl = jax.lax.dot_general(comb, U, (((1,), (0,)), ((), ())),
                               preferred_element_type=jnp.float32)
    tot = excl + carry_ref[:, :1]
    rank0 = jnp.sum(ex0f * tot, axis=0, keepdims=True).astype(jnp.int32)
    rank1 = jnp.sum(ex1f * tot, axis=0, keepdims=True).astype(jnp.int32)
    carry_ref[:, :1] += jnp.sum(comb, axis=1, keepdims=True)

    trash = E * cap
    slot0 = i0 * cap + rank0
    slot1 = i1 * cap + rank1
    k0 = rank0 < cap
    k1 = rank1 < cap
    sc0_ref[...] = jnp.where(k0, slot0, trash).reshape(1, 1, TB)
    sc1_ref[...] = jnp.where(k1, slot1, trash).reshape(1, 1, TB)
    sg0_ref[...] = jnp.where(k0, slot0, 0).reshape(1, 1, TB)
    sg1_ref[...] = jnp.where(k1, slot1, 0).reshape(1, 1, TB)
    w0_ref[...] = jnp.where(k0, p0, 0.0).reshape(1, 1, TB)
    w1_ref[...] = jnp.where(k1, p1, 0.0).reshape(1, 1, TB)
    cnt_ref[...] = carry_ref[...]


def _pack(logits_t, cap, token_block=256):
    E, T = logits_t.shape
    TB = min(token_block, T)
    nb = T // TB
    i32 = jax.ShapeDtypeStruct((nb, 1, TB), jnp.int32)
    f32 = jax.ShapeDtypeStruct((nb, 1, TB), jnp.float32)
    ospec = pl.BlockSpec((1, 1, TB), lambda i: (i, 0, 0))
    outs = pl.pallas_call(
        functools.partial(_pack_body, cap, E, TB),
        grid=(nb,),
        in_specs=[pl.BlockSpec((1, E, TB), lambda i: (0, 0, i))],
        out_specs=[ospec] * 6 + [pl.BlockSpec((E, 128), lambda i: (0, 0))],
        out_shape=[i32, i32, i32, i32, f32, f32,
                   jax.ShapeDtypeStruct((E, 128), jnp.float32)],
        scratch_shapes=[pltpu.VMEM((E, 128), jnp.float32)],
        compiler_params=pltpu.CompilerParams(
            dimension_semantics=("arbitrary",)),
    )(logits_t.reshape(1, E, T))
    return [o.reshape(T) for o in outs[:6]] + [outs[6][:, 0].astype(jnp.int32)]


def _invert_slots(sc0, sc1, n_slots):
    T = sc0.shape[0]
    n_pad = n_slots + 8
    mesh = plsc.VectorSubcoreMesh(core_axis_name="c", subcore_axis_name="s")
    nw = 32
    per = T // nw

    @functools.partial(
        pl.kernel, mesh=mesh,
        out_type=jax.ShapeDtypeStruct((n_pad,), jnp.int32),
        scratch_types=[
            pltpu.VMEM((per,), jnp.int32),
            pltpu.VMEM((per,), jnp.int32),
            pltpu.SemaphoreType.DMA,
        ],
    )
    def k(sc0_hbm, sc1_hbm, out_hbm, idx_v, val_v, sem):
        wid = lax.axis_index("s") * 2 + lax.axis_index("c")
        base = wid * per
        for i in range(per // 16):
            val_v[pl.ds(i * 16, 16)] = lax.iota(jnp.int32, 16) + (base + i * 16)
        pltpu.sync_copy(sc0_hbm.at[pl.ds(base, per)], idx_v)
        pltpu.async_copy(val_v, out_hbm.at[idx_v], sem).wait()
        pltpu.sync_copy(sc1_hbm.at[pl.ds(base, per)], idx_v)
        pltpu.async_copy(val_v, out_hbm.at[idx_v], sem).wait()

    return k(sc0, sc1)[:n_slots]


def _ffn_body(x_ref, w1_ref, w2_ref, o_ref):
    j = pl.program_id(1)
    xb = x_ref[0].astype(jnp.bfloat16)
    w1 = w1_ref[0].astype(jnp.bfloat16)
    w2 = w2_ref[0].astype(jnp.bfloat16)
    h = jax.lax.dot_general(xb, w1, (((1,), (1,)), ((), ())),
                            preferred_element_type=jnp.float32)
    h = 0.5 * h * (1.0 + jax.lax.erf(h * 0.7071067811865476))
    y = jax.lax.dot_general(h.astype(jnp.bfloat16), w2,
                            (((1,), (1,)), ((), ())),
                            preferred_element_type=jnp.float32)

    @pl.when(j == 0)
    def _init():
        o_ref[0] = y

    @pl.when(j != 0)
    def _acc():
        o_ref[0] += y


def _expert_ffn(counts, xbuf, fc1, fc2, block_ff=512):
    del counts
    E, cap, D = xbuf.shape
    DFF = fc1.shape[1]
    block_ff = min(block_ff, DFF)
    nj = DFF // block_ff
    return pl.pallas_call(
        _ffn_body,
        grid=(E, nj),
        in_specs=[
            pl.BlockSpec((1, cap, D), lambda e, j: (e, 0, 0)),
            pl.BlockSpec((1, block_ff, D), lambda e, j: (e, j, 0)),
            pl.BlockSpec((1, D, block_ff), lambda e, j: (e, 0, j)),
        ],
        out_specs=pl.BlockSpec((1, cap, D), lambda e, j: (e, 0, 0)),
        out_shape=jax.ShapeDtypeStruct((E, cap, D), jnp.float32),
        compiler_params=pltpu.CompilerParams(
            dimension_semantics=("arbitrary", "arbitrary")),
    )(xbuf, fc1, fc2)


def kernel(x, W_router, fc1, fc2):
    T, D = x.shape
    E = W_router.shape[0]
    S = T * _TOP_K
    cap = max(1, math.ceil(S * _CAP_FACTOR / E))

    logits = x @ W_router.T
    sc0, sc1, sg0, sg1, w0, w1, counts = _pack(logits.T, cap)

    src = _invert_slots(sc0, sc1, E * cap)
    xbuf = x[src].reshape(E, cap, D)

    y = _expert_ffn(counts, xbuf, fc1, fc2).reshape(E * cap, D)

    return y[sg0] * w0[:, None] + y[sg1] * w1[:, None]


# speedup vs baseline: 1.3674x; 1.0938x over previous
import functools
import math

import jax
import jax.numpy as jnp
from jax import lax
from jax.experimental import pallas as pl
from jax.experimental.pallas import tpu as pltpu
from jax.experimental.pallas import tpu_sc as plsc

_TOP_K = 2
_CAP_FACTOR = 1.25


def _pack_body(cap, E, TB, lg_ref, sc0_ref, sc1_ref, sg0_ref, sg1_ref,
               w0_ref, w1_ref, cnt_ref, carry_ref):
    i = pl.program_id(0)

    @pl.when(i == 0)
    def _init():
        carry_ref[...] = jnp.zeros_like(carry_ref)

    lg = lg_ref[0]
    sub = jax.lax.broadcasted_iota(jnp.int32, (E, TB), 0)

    v0 = jnp.max(lg, axis=0, keepdims=True)
    i0 = jnp.min(jnp.where(lg == v0, sub, E), axis=0, keepdims=True)
    ex0 = (sub == i0)
    lg1 = jnp.where(ex0, -jnp.inf, lg)
    v1 = jnp.max(lg1, axis=0, keepdims=True)
    i1 = jnp.min(jnp.where(lg1 == v1, sub, E), axis=0, keepdims=True)
    ex1 = (sub == i1)

    m = jnp.maximum(v0, v1)
    e0 = jnp.exp(v0 - m)
    e1 = jnp.exp(v1 - m)
    den = e0 + e1 + 1e-12
    p0 = e0 / den
    p1 = e1 / den

    ex0f = ex0.astype(jnp.float32)
    ex1f = ex1.astype(jnp.float32)
    comb = ex0f + ex1f
    r = jax.lax.broadcasted_iota(jnp.int32, (TB, TB), 0)
    c = jax.lax.broadcasted_iota(jnp.int32, (TB, TB), 1)
    U = (r < c).astype(jnp.float32)
    excl = jax.lax.dot_general(comb, U, (((1,), (0,)), ((), ())),
                               preferred_element_type=jnp.float32)
    tot = excl + carry_ref[:, :1]
    rank0 = jnp.sum(ex0f * tot, axis=0, keepdims=True).astype(jnp.int32)
    rank1 = jnp.sum(ex1f * tot, axis=0, keepdims=True).astype(jnp.int32)
    carry_ref[:, :1] += jnp.sum(comb, axis=1, keepdims=True)

    trash = E * cap
    slot0 = i0 * cap + rank0
    slot1 = i1 * cap + rank1
    k0 = rank0 < cap
    k1 = rank1 < cap
    sc0_ref[...] = jnp.where(k0, slot0, trash).reshape(1, 1, TB)
    sc1_ref[...] = jnp.where(k1, slot1, trash).reshape(1, 1, TB)
    sg0_ref[...] = jnp.where(k0, slot0, 0).reshape(1, 1, TB)
    sg1_ref[...] = jnp.where(k1, slot1, 0).reshape(1, 1, TB)
    w0_ref[...] = jnp.where(k0, p0, 0.0).reshape(1, 1, TB)
    w1_ref[...] = jnp.where(k1, p1, 0.0).reshape(1, 1, TB)
    cnt_ref[...] = carry_ref[...]


def _pack(logits_t, cap, token_block=256):
    E, T = logits_t.shape
    TB = min(token_block, T)
    nb = T // TB
    i32 = jax.ShapeDtypeStruct((nb, 1, TB), jnp.int32)
    f32 = jax.ShapeDtypeStruct((nb, 1, TB), jnp.float32)
    ospec = pl.BlockSpec((1, 1, TB), lambda i: (i, 0, 0))
    outs = pl.pallas_call(
        functools.partial(_pack_body, cap, E, TB),
        grid=(nb,),
        in_specs=[pl.BlockSpec((1, E, TB), lambda i: (0, 0, i))],
        out_specs=[ospec] * 6 + [pl.BlockSpec((E, 128), lambda i: (0, 0))],
        out_shape=[i32, i32, i32, i32, f32, f32,
                   jax.ShapeDtypeStruct((E, 128), jnp.float32)],
        scratch_shapes=[pltpu.VMEM((E, 128), jnp.float32)],
        compiler_params=pltpu.CompilerParams(
            dimension_semantics=("arbitrary",)),
    )(logits_t.reshape(1, E, T))
    return [o.reshape(T) for o in outs[:6]] + [outs[6][:, 0].astype(jnp.int32)]


def _invert_slots(sc0, sc1, n_slots):
    T = sc0.shape[0]
    n_pad = n_slots + 8
    mesh = plsc.VectorSubcoreMesh(core_axis_name="c", subcore_axis_name="s")
    nw = 32
    per = T // nw

    @functools.partial(
        pl.kernel, mesh=mesh,
        out_type=jax.ShapeDtypeStruct((n_pad,), jnp.int32),
        scratch_types=[
            pltpu.VMEM((per,), jnp.int32),
            pltpu.VMEM((per,), jnp.int32),
            pltpu.SemaphoreType.DMA,
        ],
    )
    def k(sc0_hbm, sc1_hbm, out_hbm, idx_v, val_v, sem):
        wid = lax.axis_index("s") * 2 + lax.axis_index("c")
        base = wid * per
        for i in range(per // 16):
            val_v[pl.ds(i * 16, 16)] = lax.iota(jnp.int32, 16) + (base + i * 16)
        pltpu.sync_copy(sc0_hbm.at[pl.ds(base, per)], idx_v)
        pltpu.async_copy(val_v, out_hbm.at[idx_v], sem).wait()
        pltpu.sync_copy(sc1_hbm.at[pl.ds(base, per)], idx_v)
        pltpu.async_copy(val_v, out_hbm.at[idx_v], sem).wait()

    return k(sc0, sc1)[:n_slots]


def _ffn_body(x_ref, w1_ref, w2_ref, o_ref):
    j = pl.program_id(1)
    xb = x_ref[0]
    w1 = w1_ref[0].astype(jnp.bfloat16)
    w2 = w2_ref[0].astype(jnp.bfloat16)
    h = jax.lax.dot_general(xb, w1, (((1,), (1,)), ((), ())),
                            preferred_element_type=jnp.float32)
    h = 0.5 * h * (1.0 + jax.lax.erf(h * 0.7071067811865476))
    y = jax.lax.dot_general(h.astype(jnp.bfloat16), w2,
                            (((1,), (1,)), ((), ())),
                            preferred_element_type=jnp.float32)

    @pl.when(j == 0)
    def _init():
        o_ref[0] = y

    @pl.when(j != 0)
    def _acc():
        o_ref[0] += y


def _expert_ffn(counts, xbuf, fc1, fc2, block_ff=1024):
    del counts
    E, cap, D = xbuf.shape
    DFF = fc1.shape[1]
    block_ff = min(block_ff, DFF)
    nj = DFF // block_ff
    return pl.pallas_call(
        _ffn_body,
        grid=(E, nj),
        in_specs=[
            pl.BlockSpec((1, cap, D), lambda e, j: (e, 0, 0)),
            pl.BlockSpec((1, block_ff, D), lambda e, j: (e, j, 0)),
            pl.BlockSpec((1, D, block_ff), lambda e, j: (e, 0, j)),
        ],
        out_specs=pl.BlockSpec((1, cap, D), lambda e, j: (e, 0, 0)),
        out_shape=jax.ShapeDtypeStruct((E, cap, D), jnp.float32),
        compiler_params=pltpu.CompilerParams(
            dimension_semantics=("arbitrary", "arbitrary")),
    )(xbuf, fc1, fc2)


def kernel(x, W_router, fc1, fc2):
    T, D = x.shape
    E = W_router.shape[0]
    S = T * _TOP_K
    cap = max(1, math.ceil(S * _CAP_FACTOR / E))

    logits = x @ W_router.T
    sc0, sc1, sg0, sg1, w0, w1, counts = _pack(logits.T, cap)

    src = _invert_slots(sc0, sc1, E * cap)
    xbuf = x.astype(jnp.bfloat16)[src].reshape(E, cap, D)

    y = _expert_ffn(counts, xbuf, fc1, fc2).reshape(E * cap, D)

    return y[sg0] * w0[:, None] + y[sg1] * w1[:, None]
